# baseline (device time: 40781 ns/iter reference)
import jax
import jax.numpy as jnp
from jax import lax
from jax.experimental import pallas as pl
from jax.experimental.pallas import tpu as pltpu

N_DEV = 8
ORDERS = (("z", "y", "x"), ("y", "x", "z"), ("x", "z", "y"))
WIDTHS = (704, 640, 704)
COL0S = (0, 704, 1344)


def kernel(x, w_mat):
    m, k_loc = x.shape
    n = w_mat.shape[1]
    chunk = m // N_DEV

    def body(x_ref, w_ref, out_ref, xb_ref, wb_ref, *scr):
        my = lax.axis_index("i")
        q = my % 4
        by = q // 2
        bx = (q % 2) ^ by
        bz = my // 4
        mybits = {"x": bx, "y": by, "z": bz}

        def pos_from(bits):
            return 4 * bits["z"] + 2 * bits["y"] + (bits["x"] ^ bits["y"])

        def partner(dim):
            b = dict(mybits)
            b[dim] = 1 - b[dim]
            return pos_from(b)

        def part(opos, c0, w):
            return jnp.dot(
                xb_ref[pl.ds(opos * chunk, chunk), :], wb_ref[:, c0:c0 + w],
                preferred_element_type=jnp.float32,
            )

        def bits_t(t):
            da, db, dc = ORDERS[t]
            return mybits[da], mybits[db], mybits[dc]

        def owner(t, a_bit, b_bit, c_bit):
            da, db, dc = ORDERS[t]
            return pos_from({da: a_bit, db: b_bit, dc: c_bit})

        barrier_sem = pltpu.get_barrier_semaphore()
        for dim in ("x", "y", "z"):
            pl.semaphore_signal(
                barrier_sem, inc=1,
                device_id=(partner(dim),), device_id_type=pl.DeviceIdType.MESH,
            )
        pl.semaphore_wait(barrier_sem, 3)

        xb_ref[:, :] = x_ref[:, :].astype(jnp.bfloat16)
        wb_ref[:, :] = w_ref[:, :].astype(jnp.bfloat16)

        T = [scr[8 * t: 8 * t + 8] for t in range(3)]
        rdmas = {}

        def exchange(t, sem_idx, src, dst, dim):
            ssem, rsem = T[t][6], T[t][7]
            rdma = pltpu.make_async_remote_copy(
                src_ref=src, dst_ref=dst,
                send_sem=ssem.at[sem_idx], recv_sem=rsem.at[sem_idx],
                device_id=(partner(dim),),
                device_id_type=pl.DeviceIdType.MESH,
            )
            rdma.start()
            rdmas[(t, sem_idx)] = rdma

        for p in range(4):
            for t in range(3):
                da, db, dc = ORDERS[t]
                ba, bb, bc = bits_t(t)
                c0, w = COL0S[t], WIDTHS[t]
                s1, r1 = T[t][0], T[t][1]
                ob = (1 - bb) if p < 2 else bb
                oc = (1 - bc) if p % 2 == 0 else bc
                opos = owner(t, 1 - ba, ob, oc)
                s1[p * chunk:(p + 1) * chunk, :] = part(opos, c0, w).astype(
                    jnp.bfloat16)
                exchange(t, p, s1.at[pl.ds(p * chunk, chunk)],
                         r1.at[pl.ds(p * chunk, chunk)], da)

        for t in range(3):
            ba, bb, bc = bits_t(t)
            c0, w = COL0S[t], WIDTHS[t]
            s2 = T[t][2]
            for p in range(2):
                oc = (1 - bc) if p == 0 else bc
                opos = owner(t, ba, 1 - bb, oc)
                s2[p * chunk:(p + 1) * chunk, :] = part(opos, c0, w).astype(
                    jnp.bfloat16)

        for p in range(2):
            for t in range(3):
                da, db, dc = ORDERS[t]
                s2, r2 = T[t][2], T[t][3]
                r1 = T[t][1]
                rdmas[(t, p)].wait_recv()
                s2[p * chunk:(p + 1) * chunk, :] = (
                    s2[p * chunk:(p + 1) * chunk, :].astype(jnp.float32)
                    + r1[p * chunk:(p + 1) * chunk, :].astype(jnp.float32)
                ).astype(jnp.bfloat16)
                exchange(t, 4 + p, s2.at[pl.ds(p * chunk, chunk)],
                         r2.at[pl.ds(p * chunk, chunk)], db)

        for t in range(3):
            ba, bb, bc = bits_t(t)
            c0, w = COL0S[t], WIDTHS[t]
            s3 = T[t][4]
            s3[:, :] = part(owner(t, ba, bb, 1 - bc), c0, w).astype(jnp.bfloat16)
            out_ref[:, c0:c0 + w] = part(my, c0, w)

        for t in range(3):
            da, db, dc = ORDERS[t]
            r1, r2, s3, r3 = T[t][1], T[t][3], T[t][4], T[t][5]
            rdmas[(t, 2)].wait_recv()
            rdmas[(t, 4)].wait_recv()
            s3[:, :] = (
                s3[:, :].astype(jnp.float32)
                + r1[2 * chunk:3 * chunk, :].astype(jnp.float32)
                + r2[0 * chunk:1 * chunk, :].astype(jnp.float32)
            ).astype(jnp.bfloat16)
            exchange(t, 6, s3, r3, dc)

        for t in range(3):
            c0, w = COL0S[t], WIDTHS[t]
            r1, r2, r3 = T[t][1], T[t][3], T[t][5]
            rdmas[(t, 3)].wait_recv()
            rdmas[(t, 5)].wait_recv()
            rdmas[(t, 6)].wait_recv()
            y = (
                out_ref[:, c0:c0 + w]
                + r1[3 * chunk:4 * chunk, :].astype(jnp.float32)
                + r2[1 * chunk:2 * chunk, :].astype(jnp.float32)
                + r3[:, :].astype(jnp.float32)
            )
            out_ref[:, c0:c0 + w] = y * jax.nn.sigmoid(y)

        for t in range(3):
            for idx in range(7):
                rdmas[(t, idx)].wait_send()

    scratch_shapes = [
        pltpu.VMEM((m, k_loc), jnp.bfloat16),
        pltpu.VMEM((k_loc, n), jnp.bfloat16),
    ]
    for t in range(3):
        w = WIDTHS[t]
        scratch_shapes += [
            pltpu.VMEM((4 * chunk, w), jnp.bfloat16),
            pltpu.VMEM((4 * chunk, w), jnp.bfloat16),
            pltpu.VMEM((2 * chunk, w), jnp.bfloat16),
            pltpu.VMEM((2 * chunk, w), jnp.bfloat16),
            pltpu.VMEM((chunk, w), jnp.bfloat16),
            pltpu.VMEM((chunk, w), jnp.bfloat16),
            pltpu.SemaphoreType.DMA((7,)),
            pltpu.SemaphoreType.DMA((7,)),
        ]

    return pl.pallas_call(
        body,
        out_shape=jax.ShapeDtypeStruct((chunk, n), jnp.float32),
        in_specs=[
            pl.BlockSpec(memory_space=pltpu.VMEM),
            pl.BlockSpec(memory_space=pltpu.VMEM),
        ],
        out_specs=pl.BlockSpec(memory_space=pltpu.VMEM),
        scratch_shapes=scratch_shapes,
        compiler_params=pltpu.CompilerParams(collective_id=0),
    )(x, w_mat)


# device time: 40269 ns/iter; 1.0127x vs baseline; 1.0127x over previous
import jax
import jax.numpy as jnp
from jax import lax
from jax.experimental import pallas as pl
from jax.experimental.pallas import tpu as pltpu

N_DEV = 8
_ROT = (("z", "y", "x"), ("y", "x", "z"), ("x", "z", "y"))
ORDERS = _ROT + _ROT
WIDTHS = (384, 384, 384, 320, 256, 320)
COL0S = (0, 384, 768, 1152, 1472, 1728)


def kernel(x, w_mat):
    m, k_loc = x.shape
    n = w_mat.shape[1]
    chunk = m // N_DEV

    def body(x_ref, w_ref, out_ref, xb_ref, wb_ref, *scr):
        my = lax.axis_index("i")
        q = my % 4
        by = q // 2
        bx = (q % 2) ^ by
        bz = my // 4
        mybits = {"x": bx, "y": by, "z": bz}

        def pos_from(bits):
            return 4 * bits["z"] + 2 * bits["y"] + (bits["x"] ^ bits["y"])

        def partner(dim):
            b = dict(mybits)
            b[dim] = 1 - b[dim]
            return pos_from(b)

        def part(opos, c0, w):
            return jnp.dot(
                xb_ref[pl.ds(opos * chunk, chunk), :], wb_ref[:, c0:c0 + w],
                preferred_element_type=jnp.float32,
            )

        def bits_t(t):
            da, db, dc = ORDERS[t]
            return mybits[da], mybits[db], mybits[dc]

        def owner(t, a_bit, b_bit, c_bit):
            da, db, dc = ORDERS[t]
            return pos_from({da: a_bit, db: b_bit, dc: c_bit})

        barrier_sem = pltpu.get_barrier_semaphore()
        for dim in ("x", "y", "z"):
            pl.semaphore_signal(
                barrier_sem, inc=1,
                device_id=(partner(dim),), device_id_type=pl.DeviceIdType.MESH,
            )
        pl.semaphore_wait(barrier_sem, 3)

        xb_ref[:, :] = x_ref[:, :].astype(jnp.bfloat16)
        wb_ref[:, :] = w_ref[:, :].astype(jnp.bfloat16)

        T = [scr[8 * t: 8 * t + 8] for t in range(len(ORDERS))]
        rdmas = {}

        def exchange(t, sem_idx, src, dst, dim):
            ssem, rsem = T[t][6], T[t][7]
            rdma = pltpu.make_async_remote_copy(
                src_ref=src, dst_ref=dst,
                send_sem=ssem.at[sem_idx], recv_sem=rsem.at[sem_idx],
                device_id=(partner(dim),),
                device_id_type=pl.DeviceIdType.MESH,
            )
            rdma.start()
            rdmas[(t, sem_idx)] = rdma

        for p in range(4):
            for t in range(len(ORDERS)):
                da, db, dc = ORDERS[t]
                ba, bb, bc = bits_t(t)
                c0, w = COL0S[t], WIDTHS[t]
                s1, r1 = T[t][0], T[t][1]
                ob = (1 - bb) if p < 2 else bb
                oc = (1 - bc) if p % 2 == 0 else bc
                opos = owner(t, 1 - ba, ob, oc)
                s1[p * chunk:(p + 1) * chunk, :] = part(opos, c0, w).astype(
                    jnp.bfloat16)
                exchange(t, p, s1.at[pl.ds(p * chunk, chunk)],
                         r1.at[pl.ds(p * chunk, chunk)], da)

        for t in range(len(ORDERS)):
            ba, bb, bc = bits_t(t)
            c0, w = COL0S[t], WIDTHS[t]
            s2 = T[t][2]
            for p in range(2):
                oc = (1 - bc) if p == 0 else bc
                opos = owner(t, ba, 1 - bb, oc)
                s2[p * chunk:(p + 1) * chunk, :] = part(opos, c0, w).astype(
                    jnp.bfloat16)

        for p in range(2):
            for t in range(len(ORDERS)):
                da, db, dc = ORDERS[t]
                s2, r2 = T[t][2], T[t][3]
                r1 = T[t][1]
                rdmas[(t, p)].wait_recv()
                s2[p * chunk:(p + 1) * chunk, :] = (
                    s2[p * chunk:(p + 1) * chunk, :].astype(jnp.float32)
                    + r1[p * chunk:(p + 1) * chunk, :].astype(jnp.float32)
                ).astype(jnp.bfloat16)
                exchange(t, 4 + p, s2.at[pl.ds(p * chunk, chunk)],
                         r2.at[pl.ds(p * chunk, chunk)], db)

        for t in range(len(ORDERS)):
            ba, bb, bc = bits_t(t)
            c0, w = COL0S[t], WIDTHS[t]
            s3 = T[t][4]
            s3[:, :] = part(owner(t, ba, bb, 1 - bc), c0, w).astype(jnp.bfloat16)
            out_ref[:, c0:c0 + w] = part(my, c0, w)

        for t in range(len(ORDERS)):
            da, db, dc = ORDERS[t]
            r1, r2, s3, r3 = T[t][1], T[t][3], T[t][4], T[t][5]
            rdmas[(t, 2)].wait_recv()
            rdmas[(t, 4)].wait_recv()
            s3[:, :] = (
                s3[:, :].astype(jnp.float32)
                + r1[2 * chunk:3 * chunk, :].astype(jnp.float32)
                + r2[0 * chunk:1 * chunk, :].astype(jnp.float32)
            ).astype(jnp.bfloat16)
            exchange(t, 6, s3, r3, dc)

        for t in range(len(ORDERS)):
            c0, w = COL0S[t], WIDTHS[t]
            r1, r2, r3 = T[t][1], T[t][3], T[t][5]
            rdmas[(t, 3)].wait_recv()
            rdmas[(t, 5)].wait_recv()
            rdmas[(t, 6)].wait_recv()
            y = (
                out_ref[:, c0:c0 + w]
                + r1[3 * chunk:4 * chunk, :].astype(jnp.float32)
                + r2[1 * chunk:2 * chunk, :].astype(jnp.float32)
                + r3[:, :].astype(jnp.float32)
            )
            out_ref[:, c0:c0 + w] = y * jax.nn.sigmoid(y)

        for t in range(len(ORDERS)):
            for idx in range(7):
                rdmas[(t, idx)].wait_send()

    scratch_shapes = [
        pltpu.VMEM((m, k_loc), jnp.bfloat16),
        pltpu.VMEM((k_loc, n), jnp.bfloat16),
    ]
    for t in range(len(ORDERS)):
        w = WIDTHS[t]
        scratch_shapes += [
            pltpu.VMEM((4 * chunk, w), jnp.bfloat16),
            pltpu.VMEM((4 * chunk, w), jnp.bfloat16),
            pltpu.VMEM((2 * chunk, w), jnp.bfloat16),
            pltpu.VMEM((2 * chunk, w), jnp.bfloat16),
            pltpu.VMEM((chunk, w), jnp.bfloat16),
            pltpu.VMEM((chunk, w), jnp.bfloat16),
            pltpu.SemaphoreType.DMA((7,)),
            pltpu.SemaphoreType.DMA((7,)),
        ]

    return pl.pallas_call(
        body,
        out_shape=jax.ShapeDtypeStruct((chunk, n), jnp.float32),
        in_specs=[
            pl.BlockSpec(memory_space=pltpu.VMEM),
            pl.BlockSpec(memory_space=pltpu.VMEM),
        ],
        out_specs=pl.BlockSpec(memory_space=pltpu.VMEM),
        scratch_shapes=scratch_shapes,
        compiler_params=pltpu.CompilerParams(collective_id=0),
    )(x, w_mat)
